# Initial kernel scaffold; baseline (speedup 1.0000x reference)
import jax
import jax.numpy as jnp
from jax.experimental import pallas as pl


def kernel(x, edge_index, W, b, W2, b2):
    raise NotImplementedError("write your pallas kernel here")


# speedup vs baseline: 113.1863x; 113.1863x over previous
import functools

import jax
import jax.numpy as jnp
from jax import lax
from jax.experimental import pallas as pl
from jax.experimental.pallas import tpu as pltpu
from jax.experimental.pallas import tpu_sc as plsc

N_NODES = 10000
N_EDGES = 320000
D_FEAT = 128
N_LABELS = 10

NS = 16
NC = 2
L = 16
N_PAD = 10240
NODES_PER_TILE = N_PAD // NS
E_HIST = N_EDGES // NS
E_PROP = N_EDGES // (NS * NC)


def _rsqrt16(d):
    i = plsc.bitcast(d, jnp.int32)
    i = jnp.int32(0x5F3759DF) - lax.shift_right_logical(i, 1)
    y = plsc.bitcast(i, jnp.float32)
    for _ in range(3):
        y = y * (jnp.float32(1.5) - jnp.float32(0.5) * d * y * y)
    return y


def _sc_body(dst_hbm, src_hbm, zeros_hbm, dinv_out, t_out,
             ebuf, sbuf, dbuf, acc, dinv_v, red_v, hist_s, dinv_s):
    c = lax.axis_index("c")
    t = lax.axis_index("s")
    zeros16 = jnp.zeros((L,), jnp.float32)

    pltpu.sync_copy(dst_hbm.at[pl.ds(t * E_HIST, E_HIST)], ebuf)
    pltpu.sync_copy(zeros_hbm, acc)

    def hist_step(i, _):
        idx = ebuf[pl.ds(i * L, L)]
        plsc.addupdate_scatter(acc, [idx], jnp.ones((L,), jnp.float32))
        return 0

    lax.fori_loop(0, E_HIST // L, hist_step, 0)
    pltpu.sync_copy(acc, hist_s.at[t])
    plsc.subcore_barrier()

    base = t * NODES_PER_TILE
    pltpu.sync_copy(hist_s.at[:, pl.ds(base, NODES_PER_TILE)], red_v)

    def dinv_step(j, _):
        deg = jnp.ones((L,), jnp.float32)

        def add_row(r, a):
            return a + red_v[r, pl.ds(j * L, L)]

        deg = lax.fori_loop(0, NS, add_row, deg)
        acc[pl.ds(j * L, L)] = _rsqrt16(deg)
        return 0

    lax.fori_loop(0, NODES_PER_TILE // L, dinv_step, 0)
    pltpu.sync_copy(acc.at[pl.ds(0, NODES_PER_TILE)],
                    dinv_s.at[pl.ds(base, NODES_PER_TILE)])
    plsc.subcore_barrier()
    pltpu.sync_copy(dinv_s, dinv_v)

    ebase = c * (NS * E_PROP) + t * E_PROP
    pltpu.sync_copy(src_hbm.at[pl.ds(ebase, E_PROP)], sbuf)
    pltpu.sync_copy(dst_hbm.at[pl.ds(ebase, E_PROP)], dbuf)
    pltpu.sync_copy(zeros_hbm, acc)

    def prop_step(i, _):
        d_idx = dbuf[pl.ds(i * L, L)]
        s_idx = sbuf[pl.ds(i * L, L)]
        vals = plsc.load_gather(dinv_v, [d_idx])
        plsc.addupdate_scatter(acc, [s_idx], vals)
        return 0

    lax.fori_loop(0, E_PROP // L, prop_step, 0)
    pltpu.sync_copy(acc, hist_s.at[t])
    plsc.subcore_barrier()

    pltpu.sync_copy(hist_s.at[:, pl.ds(base, NODES_PER_TILE)], red_v)

    def t_step(j, _):
        tv = zeros16

        def add_row(r, a):
            return a + red_v[r, pl.ds(j * L, L)]

        tv = lax.fori_loop(0, NS, add_row, tv)
        acc[pl.ds(j * L, L)] = tv
        return 0

    lax.fori_loop(0, NODES_PER_TILE // L, t_step, 0)
    pltpu.sync_copy(acc.at[pl.ds(0, NODES_PER_TILE)],
                    t_out.at[c, pl.ds(base, NODES_PER_TILE)])

    @pl.when(c == 0)
    def _():
        pltpu.sync_copy(dinv_v.at[pl.ds(base, NODES_PER_TILE)],
                        dinv_out.at[pl.ds(base, NODES_PER_TILE)])


_sc_edges = pl.kernel(
    _sc_body,
    out_type=(
        jax.ShapeDtypeStruct((N_PAD,), jnp.float32),
        jax.ShapeDtypeStruct((NC, N_PAD), jnp.float32),
    ),
    mesh=plsc.VectorSubcoreMesh(core_axis_name="c", subcore_axis_name="s",
                                num_cores=NC, num_subcores=NS),
    scratch_types=[
        pltpu.VMEM((E_HIST,), jnp.int32),
        pltpu.VMEM((E_PROP,), jnp.int32),
        pltpu.VMEM((E_PROP,), jnp.int32),
        pltpu.VMEM((N_PAD,), jnp.float32),
        pltpu.VMEM((N_PAD,), jnp.float32),
        pltpu.VMEM((NS, NODES_PER_TILE), jnp.float32),
        pltpu.VMEM_SHARED((NS, N_PAD), jnp.float32),
        pltpu.VMEM_SHARED((N_PAD,), jnp.float32),
    ],
    compiler_params=pltpu.CompilerParams(needs_layout_passes=False),
)


def _tc_body(dinv_ref, t_ref, x_ref, W_ref, b_ref, W2_ref, b2_ref, out_ref):
    dinv = dinv_ref[...]
    w = dinv * (dinv + t_ref[0:1, :] + t_ref[1:2, :])
    wm = w[:, :N_NODES]
    h = jnp.dot(x_ref[...], W_ref[...], preferred_element_type=jnp.float32)
    pooled = jnp.dot(wm, h, preferred_element_type=jnp.float32,
                     precision=jax.lax.Precision.HIGHEST)
    pooled = pooled + jnp.float32(N_NODES) * b_ref[...]
    z = jnp.dot(pooled, W2_ref[...], preferred_element_type=jnp.float32)
    out_ref[...] = jax.nn.sigmoid(z + b2_ref[...])


_tc_tail = pl.pallas_call(
    _tc_body,
    out_shape=jax.ShapeDtypeStruct((1, N_LABELS), jnp.float32),
)


@jax.jit
def kernel(x, edge_index, W, b, W2, b2):
    ei = edge_index.astype(jnp.int32)
    src = ei[0]
    dst = ei[1]
    zeros = jnp.zeros((N_PAD,), jnp.float32)
    dinv, t_part = _sc_edges(dst, src, zeros)
    out = _tc_tail(dinv.reshape(1, N_PAD), t_part, x, W,
                   b.reshape(1, -1), W2, b2.reshape(1, -1))
    return out[0]


# speedup vs baseline: 141.8405x; 1.2532x over previous
import functools

import jax
import jax.numpy as jnp
from jax import lax
from jax.experimental import pallas as pl
from jax.experimental.pallas import tpu as pltpu
from jax.experimental.pallas import tpu_sc as plsc

N_NODES = 10000
N_EDGES = 320000
D_FEAT = 128
N_LABELS = 10

NS = 16
NC = 2
L = 16
N_PAD = 10240
NODES_PER_TILE = N_PAD // NS
E_HIST = N_EDGES // NS
E_PROP = N_EDGES // (NS * NC)


def _rsqrt16(d):
    i = plsc.bitcast(d, jnp.int32)
    i = jnp.int32(0x5F3759DF) - lax.shift_right_logical(i, 1)
    y = plsc.bitcast(i, jnp.float32)
    for _ in range(3):
        y = y * (jnp.float32(1.5) - jnp.float32(0.5) * d * y * y)
    return y


def _sc_body(dst_hbm, src_hbm, zeros_hbm, dinv_out, t_out,
             ebuf, sbuf, acc, dinv_v, red_v, stage, hist_s, dinv_s,
             sem_e, sem_s, sem_z):
    c = lax.axis_index("c")
    t = lax.axis_index("s")

    cp_e = pltpu.async_copy(dst_hbm.at[pl.ds(t * E_HIST, E_HIST)], ebuf, sem_e)
    cp_s = pltpu.async_copy(
        src_hbm.at[pl.ds(t * E_HIST + c * E_PROP, E_PROP)], sbuf, sem_s)
    cp_z = pltpu.async_copy(zeros_hbm, acc, sem_z)
    cp_e.wait()
    cp_z.wait()

    @plsc.parallel_loop(0, E_HIST // L, unroll=8)
    def _(i):
        idx = ebuf[pl.ds(i * L, L)]
        plsc.addupdate_scatter(acc, [idx], jnp.ones((L,), jnp.float32))

    pltpu.sync_copy(acc, hist_s.at[t])
    plsc.subcore_barrier()

    base = t * NODES_PER_TILE
    pltpu.sync_copy(hist_s.at[:, pl.ds(base, NODES_PER_TILE)], red_v)
    cp_z2 = pltpu.async_copy(zeros_hbm, acc, sem_z)

    @plsc.parallel_loop(0, NODES_PER_TILE // L, unroll=2)
    def _(j):
        deg = jnp.ones((L,), jnp.float32)
        for r in range(NS):
            deg = deg + red_v[r, pl.ds(j * L, L)]
        stage[pl.ds(j * L, L)] = _rsqrt16(deg)

    pltpu.sync_copy(stage, dinv_s.at[pl.ds(base, NODES_PER_TILE)])

    @pl.when(c == 0)
    def _():
        pltpu.sync_copy(stage, dinv_out.at[pl.ds(base, NODES_PER_TILE)])

    plsc.subcore_barrier()
    pltpu.sync_copy(dinv_s, dinv_v)
    cp_s.wait()
    cp_z2.wait()

    dbase = c * E_PROP

    @plsc.parallel_loop(0, E_PROP // L, unroll=8)
    def _(i):
        d_idx = ebuf[pl.ds(dbase + i * L, L)]
        s_idx = sbuf[pl.ds(i * L, L)]
        vals = plsc.load_gather(dinv_v, [d_idx])
        plsc.addupdate_scatter(acc, [s_idx], vals)

    pltpu.sync_copy(acc, hist_s.at[t])
    plsc.subcore_barrier()

    pltpu.sync_copy(hist_s.at[:, pl.ds(base, NODES_PER_TILE)], red_v)

    @plsc.parallel_loop(0, NODES_PER_TILE // L, unroll=2)
    def _(j):
        tv = jnp.zeros((L,), jnp.float32)
        for r in range(NS):
            tv = tv + red_v[r, pl.ds(j * L, L)]
        stage[pl.ds(j * L, L)] = tv

    pltpu.sync_copy(stage, t_out.at[c, pl.ds(base, NODES_PER_TILE)])


_sc_edges = pl.kernel(
    _sc_body,
    out_type=(
        jax.ShapeDtypeStruct((N_PAD,), jnp.float32),
        jax.ShapeDtypeStruct((NC, N_PAD), jnp.float32),
    ),
    mesh=plsc.VectorSubcoreMesh(core_axis_name="c", subcore_axis_name="s",
                                num_cores=NC, num_subcores=NS),
    scratch_types=[
        pltpu.VMEM((E_HIST,), jnp.int32),
        pltpu.VMEM((E_PROP,), jnp.int32),
        pltpu.VMEM((N_PAD,), jnp.float32),
        pltpu.VMEM((N_PAD,), jnp.float32),
        pltpu.VMEM((NS, NODES_PER_TILE), jnp.float32),
        pltpu.VMEM((NODES_PER_TILE,), jnp.float32),
        pltpu.VMEM_SHARED((NS, N_PAD), jnp.float32),
        pltpu.VMEM_SHARED((N_PAD,), jnp.float32),
        pltpu.SemaphoreType.DMA,
        pltpu.SemaphoreType.DMA,
        pltpu.SemaphoreType.DMA,
    ],
    compiler_params=pltpu.CompilerParams(needs_layout_passes=False),
)


def _tc_body(dinv_ref, t_ref, x_ref, W_ref, b_ref, W2_ref, b2_ref, out_ref):
    dinv = dinv_ref[...]
    w = dinv * (dinv + t_ref[0:1, :] + t_ref[1:2, :])
    wm = w[:, :N_NODES]
    h = jnp.dot(x_ref[...], W_ref[...], preferred_element_type=jnp.float32)
    pooled = jnp.dot(wm, h, preferred_element_type=jnp.float32,
                     precision=jax.lax.Precision.HIGHEST)
    pooled = pooled + jnp.float32(N_NODES) * b_ref[...]
    z = jnp.dot(pooled, W2_ref[...], preferred_element_type=jnp.float32)
    out_ref[...] = jax.nn.sigmoid(z + b2_ref[...])


_tc_tail = pl.pallas_call(
    _tc_body,
    out_shape=jax.ShapeDtypeStruct((1, N_LABELS), jnp.float32),
)


@jax.jit
def kernel(x, edge_index, W, b, W2, b2):
    ei = edge_index.astype(jnp.int32)
    src = ei[0]
    dst = ei[1]
    zeros = jnp.zeros((N_PAD,), jnp.float32)
    dinv, t_part = _sc_edges(dst, src, zeros)
    out = _tc_tail(dinv.reshape(1, N_PAD), t_part, x, W,
                   b.reshape(1, -1), W2, b2.reshape(1, -1))
    return out[0]


# speedup vs baseline: 176.8475x; 1.2468x over previous
import functools

import jax
import jax.numpy as jnp
from jax import lax
from jax.experimental import pallas as pl
from jax.experimental.pallas import tpu as pltpu
from jax.experimental.pallas import tpu_sc as plsc

N_NODES = 10000
N_EDGES = 320000
D_FEAT = 128
N_LABELS = 10

NS = 16
NC = 2
L = 16
N_PAD = 10240
NODES_PER_TILE = N_PAD // NS
E_HIST = N_EDGES // NS
E_PROP = N_EDGES // (NS * NC)


def _rsqrt16(d):
    i = plsc.bitcast(d, jnp.int32)
    i = jnp.int32(0x5F3759DF) - lax.shift_right_logical(i, 1)
    y = plsc.bitcast(i, jnp.float32)
    for _ in range(3):
        y = y * (jnp.float32(1.5) - jnp.float32(0.5) * d * y * y)
    return y


def _sc_body(edge_hbm, zeros_hbm, dinv_out, t_out,
             ebuf, sbuf, acc, dinv_v, red_v, stage, hist_s, dinv_s,
             sem_e, sem_s, sem_z):
    c = lax.axis_index("c")
    t = lax.axis_index("s")

    cp_e = pltpu.async_copy(
        edge_hbm.at[pl.ds(N_EDGES + t * E_HIST, E_HIST)], ebuf, sem_e)
    cp_s = pltpu.async_copy(
        edge_hbm.at[pl.ds(t * E_HIST + c * E_PROP, E_PROP)], sbuf, sem_s)
    cp_z = pltpu.async_copy(zeros_hbm, acc, sem_z)
    cp_e.wait()
    cp_z.wait()

    @plsc.parallel_loop(0, E_HIST // L, unroll=8)
    def _(i):
        idx = ebuf[pl.ds(i * L, L)]
        plsc.addupdate_scatter(acc, [idx], jnp.ones((L,), jnp.float32))

    pltpu.sync_copy(acc, hist_s.at[t])
    plsc.subcore_barrier()

    base = t * NODES_PER_TILE
    pltpu.sync_copy(hist_s.at[:, pl.ds(base, NODES_PER_TILE)], red_v)
    cp_z2 = pltpu.async_copy(zeros_hbm, acc, sem_z)

    @plsc.parallel_loop(0, NODES_PER_TILE // L, unroll=2)
    def _(j):
        deg = jnp.ones((L,), jnp.float32)
        for r in range(NS):
            deg = deg + red_v[r, pl.ds(j * L, L)]
        stage[pl.ds(j * L, L)] = _rsqrt16(deg)

    pltpu.sync_copy(stage, dinv_s.at[pl.ds(base, NODES_PER_TILE)])

    @pl.when(c == 0)
    def _():
        pltpu.sync_copy(stage, dinv_out.at[pl.ds(base, NODES_PER_TILE)])

    plsc.subcore_barrier()
    pltpu.sync_copy(dinv_s, dinv_v)
    cp_s.wait()
    cp_z2.wait()

    dbase = c * E_PROP

    @plsc.parallel_loop(0, E_PROP // L, unroll=8)
    def _(i):
        d_idx = ebuf[pl.ds(dbase + i * L, L)]
        s_idx = sbuf[pl.ds(i * L, L)]
        vals = plsc.load_gather(dinv_v, [d_idx])
        plsc.addupdate_scatter(acc, [s_idx], vals)

    pltpu.sync_copy(acc, hist_s.at[t])
    plsc.subcore_barrier()

    pltpu.sync_copy(hist_s.at[:, pl.ds(base, NODES_PER_TILE)], red_v)

    @plsc.parallel_loop(0, NODES_PER_TILE // L, unroll=2)
    def _(j):
        tv = jnp.zeros((L,), jnp.float32)
        for r in range(NS):
            tv = tv + red_v[r, pl.ds(j * L, L)]
        stage[pl.ds(j * L, L)] = tv

    pltpu.sync_copy(stage, t_out.at[c, pl.ds(base, NODES_PER_TILE)])


_sc_edges = pl.kernel(
    _sc_body,
    out_type=(
        jax.ShapeDtypeStruct((N_PAD,), jnp.float32),
        jax.ShapeDtypeStruct((NC, N_PAD), jnp.float32),
    ),
    mesh=plsc.VectorSubcoreMesh(core_axis_name="c", subcore_axis_name="s",
                                num_cores=NC, num_subcores=NS),
    scratch_types=[
        pltpu.VMEM((E_HIST,), jnp.int32),
        pltpu.VMEM((E_PROP,), jnp.int32),
        pltpu.VMEM((N_PAD,), jnp.float32),
        pltpu.VMEM((N_PAD,), jnp.float32),
        pltpu.VMEM((NS, NODES_PER_TILE), jnp.float32),
        pltpu.VMEM((NODES_PER_TILE,), jnp.float32),
        pltpu.VMEM_SHARED((NS, N_PAD), jnp.float32),
        pltpu.VMEM_SHARED((N_PAD,), jnp.float32),
        pltpu.SemaphoreType.DMA,
        pltpu.SemaphoreType.DMA,
        pltpu.SemaphoreType.DMA,
    ],
    compiler_params=pltpu.CompilerParams(needs_layout_passes=False),
)


def _tc_body(dinv_ref, t_ref, x_ref, W_ref, b_ref, W2_ref, b2_ref, out_ref):
    dinv = dinv_ref[...]
    w = dinv * (dinv + t_ref[0:1, :] + t_ref[1:2, :])
    wm = w[:, :N_NODES]
    h = jnp.dot(x_ref[...], W_ref[...], preferred_element_type=jnp.float32)
    pooled = jnp.dot(wm, h, preferred_element_type=jnp.float32,
                     precision=jax.lax.Precision.HIGHEST)
    pooled = pooled + jnp.float32(N_NODES) * b_ref[...]
    z = jnp.dot(pooled, W2_ref[...], preferred_element_type=jnp.float32)
    out_ref[...] = jax.nn.sigmoid(z + b2_ref[...])


_tc_tail = pl.pallas_call(
    _tc_body,
    out_shape=jax.ShapeDtypeStruct((1, N_LABELS), jnp.float32),
)


@jax.jit
def kernel(x, edge_index, W, b, W2, b2):
    ei = edge_index.astype(jnp.int32).reshape(-1)
    zeros = jnp.zeros((N_PAD,), jnp.float32)
    dinv, t_part = _sc_edges(ei, zeros)
    out = _tc_tail(dinv.reshape(1, N_PAD), t_part, x, W,
                   b.reshape(1, -1), W2, b2.reshape(1, -1))
    return out[0]


# speedup vs baseline: 194.4969x; 1.0998x over previous
import functools

import jax
import jax.numpy as jnp
from jax import lax
from jax.experimental import pallas as pl
from jax.experimental.pallas import tpu as pltpu
from jax.experimental.pallas import tpu_sc as plsc

N_NODES = 10000
N_EDGES = 320000
D_FEAT = 128
N_LABELS = 10

NS = 16
NC = 2
L = 16
N_PAD = 10240
NODES_PER_TILE = N_PAD // NS
E_BUF = 20096
E_LAST_BASE = N_EDGES - E_BUF
E_SKIP = 15 * E_BUF - E_LAST_BASE
E_LAST = E_BUF - E_SKIP


def _rsqrt16(d):
    i = plsc.bitcast(d, jnp.int32)
    i = jnp.int32(0x5F3759DF) - lax.shift_right_logical(i, 1)
    y = plsc.bitcast(i, jnp.float32)
    for _ in range(3):
        y = y * (jnp.float32(1.5) - jnp.float32(0.5) * d * y * y)
    return y


def _sc_body(edge_hbm, zeros_hbm, dinv_out, t_out,
             ebuf, acc, dinv_v, red_v, stage, hist_s, dinv_s,
             sem_e, sem_z):
    c = lax.axis_index("c")
    t = lax.axis_index("s")

    is_last = t == NS - 1
    ebase = jnp.where(is_last, E_LAST_BASE, t * E_BUF)
    off = jnp.where(is_last, E_SKIP, 0)
    n_mine = jnp.where(is_last, E_LAST, E_BUF)
    cp_e = pltpu.async_copy(edge_hbm.at[:, pl.ds(ebase, E_BUF)], ebuf, sem_e)
    cp_z = pltpu.async_copy(zeros_hbm, acc, sem_z)
    cp_e.wait()
    cp_z.wait()

    @plsc.parallel_loop(off // L, (off + n_mine) // L, unroll=8)
    def _(i):
        idx = ebuf[1, pl.ds(i * L, L)]
        plsc.addupdate_scatter(acc, [idx], jnp.ones((L,), jnp.float32))

    pltpu.sync_copy(acc, hist_s.at[t])
    plsc.subcore_barrier()

    base = t * NODES_PER_TILE
    pltpu.sync_copy(hist_s.at[:, pl.ds(base, NODES_PER_TILE)], red_v)
    cp_z2 = pltpu.async_copy(zeros_hbm, acc, sem_z)

    @plsc.parallel_loop(0, NODES_PER_TILE // L, unroll=2)
    def _(j):
        deg = jnp.ones((L,), jnp.float32)
        for r in range(NS):
            deg = deg + red_v[r, pl.ds(j * L, L)]
        stage[pl.ds(j * L, L)] = _rsqrt16(deg)

    pltpu.sync_copy(stage, dinv_s.at[pl.ds(base, NODES_PER_TILE)])

    @pl.when(c == 0)
    def _():
        pltpu.sync_copy(stage, dinv_out.at[pl.ds(base, NODES_PER_TILE)])

    plsc.subcore_barrier()
    pltpu.sync_copy(dinv_s, dinv_v)
    cp_z2.wait()

    half = n_mine // 2
    dlo = (off + c * half) // L
    dhi = (off + c * half + half) // L

    @plsc.parallel_loop(dlo, dhi, unroll=8)
    def _(i):
        d_idx = ebuf[1, pl.ds(i * L, L)]
        s_idx = ebuf[0, pl.ds(i * L, L)]
        vals = plsc.load_gather(dinv_v, [d_idx])
        plsc.addupdate_scatter(acc, [s_idx], vals)

    pltpu.sync_copy(acc, hist_s.at[t])
    plsc.subcore_barrier()

    pltpu.sync_copy(hist_s.at[:, pl.ds(base, NODES_PER_TILE)], red_v)

    @plsc.parallel_loop(0, NODES_PER_TILE // L, unroll=2)
    def _(j):
        tv = jnp.zeros((L,), jnp.float32)
        for r in range(NS):
            tv = tv + red_v[r, pl.ds(j * L, L)]
        stage[pl.ds(j * L, L)] = tv

    pltpu.sync_copy(stage, t_out.at[c, pl.ds(base, NODES_PER_TILE)])


_sc_edges = pl.kernel(
    _sc_body,
    out_type=(
        jax.ShapeDtypeStruct((N_PAD,), jnp.float32),
        jax.ShapeDtypeStruct((NC, N_PAD), jnp.float32),
    ),
    mesh=plsc.VectorSubcoreMesh(core_axis_name="c", subcore_axis_name="s",
                                num_cores=NC, num_subcores=NS),
    scratch_types=[
        pltpu.VMEM((2, E_BUF), jnp.int32),
        pltpu.VMEM((N_PAD,), jnp.float32),
        pltpu.VMEM((N_PAD,), jnp.float32),
        pltpu.VMEM((NS, NODES_PER_TILE), jnp.float32),
        pltpu.VMEM((NODES_PER_TILE,), jnp.float32),
        pltpu.VMEM_SHARED((NS, N_PAD), jnp.float32),
        pltpu.VMEM_SHARED((N_PAD,), jnp.float32),
        pltpu.SemaphoreType.DMA,
        pltpu.SemaphoreType.DMA,
    ],
    compiler_params=pltpu.CompilerParams(needs_layout_passes=False),
)


def _tc_body(dinv_ref, t_ref, x_ref, W_ref, b_ref, W2_ref, b2_ref, out_ref):
    dinv = dinv_ref[...]
    w = dinv * (dinv + t_ref[0:1, :] + t_ref[1:2, :])
    wm = w[:, :N_NODES]
    h = jnp.dot(x_ref[...], W_ref[...], preferred_element_type=jnp.float32)
    pooled = jnp.dot(wm, h, preferred_element_type=jnp.float32,
                     precision=jax.lax.Precision.HIGHEST)
    pooled = pooled + jnp.float32(N_NODES) * b_ref[...]
    z = jnp.dot(pooled, W2_ref[...], preferred_element_type=jnp.float32)
    out_ref[...] = jax.nn.sigmoid(z + b2_ref[...])


_tc_tail = pl.pallas_call(
    _tc_body,
    out_shape=jax.ShapeDtypeStruct((1, N_LABELS), jnp.float32),
)


@jax.jit
def kernel(x, edge_index, W, b, W2, b2):
    ei = edge_index.astype(jnp.int32)
    zeros = jnp.zeros((N_PAD,), jnp.float32)
    dinv, t_part = _sc_edges(ei, zeros)
    out = _tc_tail(dinv.reshape(1, N_PAD), t_part, x, W,
                   b.reshape(1, -1), W2, b2.reshape(1, -1))
    return out[0]


# speedup vs baseline: 210.9849x; 1.0848x over previous
import functools

import jax
import jax.numpy as jnp
from jax import lax
from jax.experimental import pallas as pl
from jax.experimental.pallas import tpu as pltpu
from jax.experimental.pallas import tpu_sc as plsc

N_NODES = 10000
N_EDGES = 320000
D_FEAT = 128
N_LABELS = 10

NS = 16
NC = 2
L = 16
N_PAD = 10240
NODES_PER_TILE = N_PAD // NS
E_BUF = 20096
E_LAST_BASE = N_EDGES - E_BUF
E_SKIP = 15 * E_BUF - E_LAST_BASE
E_LAST = E_BUF - E_SKIP


def _rsqrt16(d):
    i = plsc.bitcast(d, jnp.int32)
    i = jnp.int32(0x5F3759DF) - lax.shift_right_logical(i, 1)
    y = plsc.bitcast(i, jnp.float32)
    for _ in range(3):
        y = y * (jnp.float32(1.5) - jnp.float32(0.5) * d * y * y)
    return y


def _sc_body(edge_hbm, dinv_out, t_out,
             ebuf, acc, dinv_v, red_v, stage, hist_s, dinv_s, sem_e):
    c = lax.axis_index("c")
    t = lax.axis_index("s")

    def zero_acc():
        @plsc.parallel_loop(0, N_PAD // L, unroll=4)
        def _(j):
            acc[pl.ds(j * L, L)] = jnp.zeros((L,), jnp.float32)

    is_last = t == NS - 1
    ebase = jnp.where(is_last, E_LAST_BASE, t * E_BUF)
    off = jnp.where(is_last, E_SKIP, 0)
    n_mine = jnp.where(is_last, E_LAST, E_BUF)
    cp_e = pltpu.async_copy(edge_hbm.at[:, pl.ds(ebase, E_BUF)], ebuf, sem_e)
    zero_acc()
    cp_e.wait()

    @plsc.parallel_loop(off // L, (off + n_mine) // L, unroll=8)
    def _(i):
        idx = ebuf[1, pl.ds(i * L, L)]
        plsc.addupdate_scatter(acc, [idx], jnp.ones((L,), jnp.float32))

    pltpu.sync_copy(acc, hist_s.at[t])
    plsc.subcore_barrier()

    base = t * NODES_PER_TILE
    pltpu.sync_copy(hist_s.at[:, pl.ds(base, NODES_PER_TILE)], red_v)
    zero_acc()

    @plsc.parallel_loop(0, NODES_PER_TILE // L, unroll=2)
    def _(j):
        deg = jnp.ones((L,), jnp.float32)
        for r in range(NS):
            deg = deg + red_v[r, pl.ds(j * L, L)]
        stage[pl.ds(j * L, L)] = _rsqrt16(deg)

    pltpu.sync_copy(stage, dinv_s.at[pl.ds(base, NODES_PER_TILE)])

    @pl.when(c == 0)
    def _():
        pltpu.sync_copy(stage, dinv_out.at[pl.ds(base, NODES_PER_TILE)])

    plsc.subcore_barrier()
    pltpu.sync_copy(dinv_s, dinv_v)

    half = n_mine // 2
    dlo = (off + c * half) // L
    dhi = (off + c * half + half) // L

    @plsc.parallel_loop(dlo, dhi, unroll=8)
    def _(i):
        d_idx = ebuf[1, pl.ds(i * L, L)]
        s_idx = ebuf[0, pl.ds(i * L, L)]
        vals = plsc.load_gather(dinv_v, [d_idx])
        plsc.addupdate_scatter(acc, [s_idx], vals)

    pltpu.sync_copy(acc, hist_s.at[t])
    plsc.subcore_barrier()

    pltpu.sync_copy(hist_s.at[:, pl.ds(base, NODES_PER_TILE)], red_v)

    @plsc.parallel_loop(0, NODES_PER_TILE // L, unroll=2)
    def _(j):
        tv = jnp.zeros((L,), jnp.float32)
        for r in range(NS):
            tv = tv + red_v[r, pl.ds(j * L, L)]
        stage[pl.ds(j * L, L)] = tv

    pltpu.sync_copy(stage, t_out.at[c, pl.ds(base, NODES_PER_TILE)])


_sc_edges = pl.kernel(
    _sc_body,
    out_type=(
        jax.ShapeDtypeStruct((N_PAD,), jnp.float32),
        jax.ShapeDtypeStruct((NC, N_PAD), jnp.float32),
    ),
    mesh=plsc.VectorSubcoreMesh(core_axis_name="c", subcore_axis_name="s",
                                num_cores=NC, num_subcores=NS),
    scratch_types=[
        pltpu.VMEM((2, E_BUF), jnp.int32),
        pltpu.VMEM((N_PAD,), jnp.float32),
        pltpu.VMEM((N_PAD,), jnp.float32),
        pltpu.VMEM((NS, NODES_PER_TILE), jnp.float32),
        pltpu.VMEM((NODES_PER_TILE,), jnp.float32),
        pltpu.VMEM_SHARED((NS, N_PAD), jnp.float32),
        pltpu.VMEM_SHARED((N_PAD,), jnp.float32),
        pltpu.SemaphoreType.DMA,
    ],
    compiler_params=pltpu.CompilerParams(needs_layout_passes=False),
)


def _tc_body(dinv_ref, t_ref, x_ref, W_ref, b_ref, W2_ref, b2_ref, out_ref):
    dinv = dinv_ref[...]
    w = dinv * (dinv + t_ref[0:1, :] + t_ref[1:2, :])
    wm = w[:, :N_NODES]
    hp = jax.lax.Precision.HIGHEST
    xb = x_ref[...].astype(jnp.bfloat16).astype(jnp.float32)
    Wb = W_ref[...].astype(jnp.bfloat16).astype(jnp.float32)
    s = jnp.dot(wm, xb, preferred_element_type=jnp.float32, precision=hp)
    pooled = jnp.dot(s, Wb, preferred_element_type=jnp.float32, precision=hp)
    pooled = pooled + jnp.float32(N_NODES) * b_ref[...]
    z = jnp.dot(pooled, W2_ref[...], preferred_element_type=jnp.float32)
    out_ref[...] = jax.nn.sigmoid(z + b2_ref[...])


_tc_tail = pl.pallas_call(
    _tc_body,
    out_shape=jax.ShapeDtypeStruct((1, N_LABELS), jnp.float32),
)


@jax.jit
def kernel(x, edge_index, W, b, W2, b2):
    ei = edge_index.astype(jnp.int32)
    dinv, t_part = _sc_edges(ei)
    out = _tc_tail(dinv.reshape(1, N_PAD), t_part, x, W,
                   b.reshape(1, -1), W2, b2.reshape(1, -1))
    return out[0]
